# baseline (speedup 1.0000x reference)
import math

import jax
import jax.numpy as jnp
from jax import lax
from jax.experimental import pallas as pl

_NBINS = 10
_H = 512
_W = 512
_OUT = 507
_FR = 520
_FC = 640


def _atan2(y, x):
    ax = jnp.abs(x)
    ay = jnp.abs(y)
    mx = jnp.maximum(ax, ay)
    mn = jnp.minimum(ax, ay)
    q = mn / jnp.where(mx == 0.0, 1.0, mx)
    big = q > 0.41421356237309503
    qr = jnp.where(big, (q - 1.0) / (q + 1.0), q)
    z = qr * qr
    poly = ((8.05374449538e-2 * z - 1.38776856032e-1) * z
            + 1.99777106478e-1) * z - 3.33329491539e-1
    a = qr + qr * z * poly + jnp.where(big, 0.7853981633974483, 0.0)
    a = jnp.where(ay > ax, 1.5707963267948966 - a, a)
    a = jnp.where(x < 0.0, math.pi - a, a)
    return jnp.where(y < 0.0, -a, a)


def _hog_body(x_ref, o_ref):
    xb = x_ref[0, 0].astype(jnp.bfloat16)
    ej = lax.broadcasted_iota(jnp.int32, (_W, _W), 0)
    ec = lax.broadcasted_iota(jnp.int32, (_W, _W), 1)
    dd = ec + 1 - ej
    esm = ((dd == 0) | (dd == 2)).astype(jnp.bfloat16) + 2.0 * (dd == 1).astype(jnp.bfloat16)
    edf = (dd == 2).astype(jnp.bfloat16) - (dd == 0).astype(jnp.bfloat16)
    hsm = jnp.dot(xb, esm, preferred_element_type=jnp.float32)
    hdf = jnp.dot(xb, edf, preferred_element_type=jnp.float32)
    zrows = jnp.zeros((_FR - _H, _W), jnp.float32)
    hsm = jnp.concatenate([hsm, zrows], axis=0)
    hdf = jnp.concatenate([hdf, zrows], axis=0)
    gx = jnp.roll(hdf, 2, axis=0) + 2.0 * jnp.roll(hdf, 1, axis=0) + hdf
    gy = jnp.roll(hsm, 2, axis=0) - hsm

    mag = jnp.sqrt(gx * gx + gy * gy)
    ri = lax.broadcasted_iota(jnp.int32, (_FR, _W), 0)
    rvalid = (ri >= 1) & (ri <= _H)
    mag = jnp.where(rvalid, mag * (1.0 / 64.0), 0.0)

    p = _atan2(gx, gy) * (_NBINS / math.pi)
    fl = jnp.floor(p)
    frac = p - fl
    bq = jnp.where(fl >= 10.0, fl - 10.0, jnp.where(fl < 0.0, fl + 10.0, fl))
    tq = jnp.where(frac == 0.0, bq, jnp.where(bq == 9.0, 0.0, bq + 1.0))
    b_v = mag * (1.0 - frac)
    t_v = mag * ((1.0 + frac) - (tq - bq))

    bqh = bq.astype(jnp.bfloat16)
    tqh = tq.astype(jnp.bfloat16)
    bvh = b_v.astype(jnp.bfloat16)
    tvh = t_v.astype(jnp.bfloat16)
    zero = jnp.zeros_like(bvh)

    bc = lax.broadcasted_iota(jnp.int32, (_W, 512), 0)
    bj = lax.broadcasted_iota(jnp.int32, (_W, 512), 1)
    bmat = ((bc - bj >= -1) & (bc - bj <= 6)).astype(jnp.bfloat16)
    ai = lax.broadcasted_iota(jnp.int32, (_W, _FR), 0)
    ar = lax.broadcasted_iota(jnp.int32, (_W, _FR), 1)
    amat = ((ar - ai >= 0) & (ar - ai <= 7)).astype(jnp.bfloat16)

    for k in range(_NBINS):
        kk = jnp.bfloat16(k)
        h = jnp.where(bqh == kk, bvh, zero) + jnp.where(tqh == kk, tvh, zero)
        hv = jnp.dot(amat, h, preferred_element_type=jnp.float32)
        pooled = jnp.dot(hv.astype(jnp.bfloat16), bmat,
                         preferred_element_type=jnp.float32)
        o_ref[0, k] = pooled[0:_OUT, 0:_OUT]


def kernel(x):
    n = x.shape[0]
    return pl.pallas_call(
        _hog_body,
        grid=(n,),
        in_specs=[pl.BlockSpec((1, 1, _H, _W), lambda i: (i, 0, 0, 0))],
        out_specs=pl.BlockSpec((1, _NBINS, _OUT, _OUT), lambda i: (i, 0, 0, 0)),
        out_shape=jax.ShapeDtypeStruct((n, _NBINS, _OUT, _OUT), jnp.float32),
    )(x)


# speedup vs baseline: 1.2436x; 1.2436x over previous
import math

import jax
import jax.numpy as jnp
from jax import lax
from jax.experimental import pallas as pl

_NBINS = 10
_H = 512
_W = 512
_OUT = 507
_FR = 520
_FC = 640


def _atan2(y, x):
    ax = jnp.abs(x)
    ay = jnp.abs(y)
    mx = jnp.maximum(ax, ay)
    mn = jnp.minimum(ax, ay)
    q = mn / jnp.where(mx == 0.0, 1.0, mx)
    big = q > 0.41421356237309503
    qr = jnp.where(big, (q - 1.0) / (q + 1.0), q)
    z = qr * qr
    poly = ((8.05374449538e-2 * z - 1.38776856032e-1) * z
            + 1.99777106478e-1) * z - 3.33329491539e-1
    a = qr + qr * z * poly + jnp.where(big, 0.7853981633974483, 0.0)
    a = jnp.where(ay > ax, 1.5707963267948966 - a, a)
    a = jnp.where(x < 0.0, math.pi - a, a)
    return jnp.where(y < 0.0, -a, a)


def _hog_body(x_ref, o_ref):
    xb = x_ref[0, 0].astype(jnp.bfloat16)
    ej = lax.broadcasted_iota(jnp.int32, (_W, _W), 0)
    ec = lax.broadcasted_iota(jnp.int32, (_W, _W), 1)
    dd = ec + 1 - ej
    esm = ((dd == 0) | (dd == 2)).astype(jnp.bfloat16) + 2.0 * (dd == 1).astype(jnp.bfloat16)
    edf = (dd == 2).astype(jnp.bfloat16) - (dd == 0).astype(jnp.bfloat16)
    hsm = jnp.dot(xb, esm, preferred_element_type=jnp.float32)
    hdf = jnp.dot(xb, edf, preferred_element_type=jnp.float32)
    zrows = jnp.zeros((_FR - _H, _W), jnp.float32)
    hsm = jnp.concatenate([hsm, zrows], axis=0)
    hdf = jnp.concatenate([hdf, zrows], axis=0)
    gx = jnp.roll(hdf, 2, axis=0) + 2.0 * jnp.roll(hdf, 1, axis=0) + hdf
    gy = jnp.roll(hsm, 2, axis=0) - hsm

    mag = jnp.sqrt(gx * gx + gy * gy)
    ri = lax.broadcasted_iota(jnp.int32, (_FR, _W), 0)
    rvalid = (ri >= 1) & (ri <= _H)
    mag = jnp.where(rvalid, mag * (1.0 / 64.0), 0.0)

    p = _atan2(gx, gy) * (_NBINS / math.pi)
    fl = jnp.floor(p)
    frac = p - fl
    bq = jnp.where(fl >= 10.0, fl - 10.0, jnp.where(fl < 0.0, fl + 10.0, fl))
    tq = jnp.where(frac == 0.0, bq, jnp.where(bq == 9.0, 0.0, bq + 1.0))
    b_v = mag * (1.0 - frac)
    t_v = mag * ((1.0 + frac) - (tq - bq))

    bqh = bq.astype(jnp.bfloat16)
    tqh = tq.astype(jnp.bfloat16)
    bvh = b_v.astype(jnp.bfloat16)
    tvh = t_v.astype(jnp.bfloat16)
    zero = jnp.zeros_like(bvh)

    bc = lax.broadcasted_iota(jnp.int32, (_W, 512), 0)
    bj = lax.broadcasted_iota(jnp.int32, (_W, 512), 1)
    bmat = ((bc - bj >= -1) & (bc - bj <= 6)).astype(jnp.bfloat16)
    for k in range(_NBINS):
        kk = jnp.bfloat16(k)
        h = jnp.where(bqh == kk, bvh, zero) + jnp.where(tqh == kk, tvh, zero)
        h = h + jnp.roll(h, -1, axis=0)
        h = h + jnp.roll(h, -2, axis=0)
        h = h + jnp.roll(h, -4, axis=0)
        pooled = jnp.dot(h, bmat, preferred_element_type=jnp.float32)
        o_ref[0, k] = pooled[0:_OUT, 0:_OUT]


def kernel(x):
    n = x.shape[0]
    return pl.pallas_call(
        _hog_body,
        grid=(n,),
        in_specs=[pl.BlockSpec((1, 1, _H, _W), lambda i: (i, 0, 0, 0))],
        out_specs=pl.BlockSpec((1, _NBINS, _OUT, _OUT), lambda i: (i, 0, 0, 0)),
        out_shape=jax.ShapeDtypeStruct((n, _NBINS, _OUT, _OUT), jnp.float32),
    )(x)


# speedup vs baseline: 1.2451x; 1.0013x over previous
import math

import jax
import jax.numpy as jnp
from jax import lax
from jax.experimental import pallas as pl

_NBINS = 10
_H = 512
_W = 512
_OUT = 507
_FR = 520


def _atan2(y, x):
    ax = jnp.abs(x)
    ay = jnp.abs(y)
    mx = jnp.maximum(ax, ay)
    mn = jnp.minimum(ax, ay)
    q = mn / jnp.where(mx == 0.0, 1.0, mx)
    big = q > 0.41421356237309503
    qr = jnp.where(big, (q - 1.0) / (q + 1.0), q)
    z = qr * qr
    poly = ((8.05374449538e-2 * z - 1.38776856032e-1) * z
            + 1.99777106478e-1) * z - 3.33329491539e-1
    a = qr + qr * z * poly + jnp.where(big, 0.7853981633974483, 0.0)
    a = jnp.where(ay > ax, 1.5707963267948966 - a, a)
    a = jnp.where(x < 0.0, math.pi - a, a)
    return jnp.where(y < 0.0, -a, a)


def _hog_body(x_ref, o_ref):
    xb = x_ref[0, 0].astype(jnp.bfloat16)
    ej = lax.broadcasted_iota(jnp.int32, (_W, _W), 0)
    ec = lax.broadcasted_iota(jnp.int32, (_W, _W), 1)
    dd = ec + 1 - ej
    esm = ((dd == 0) | (dd == 2)).astype(jnp.bfloat16) + 2.0 * (dd == 1).astype(jnp.bfloat16)
    edf = (dd == 2).astype(jnp.bfloat16) - (dd == 0).astype(jnp.bfloat16)
    hsm = jnp.dot(xb, esm, preferred_element_type=jnp.float32)
    hdf = jnp.dot(xb, edf, preferred_element_type=jnp.float32)
    zrows = jnp.zeros((_FR - _H, _W), jnp.float32)
    hsm = jnp.concatenate([hsm, zrows], axis=0)
    hdf = jnp.concatenate([hdf, zrows], axis=0)
    gx = jnp.roll(hdf, 2, axis=0) + 2.0 * jnp.roll(hdf, 1, axis=0) + hdf
    gy = jnp.roll(hsm, 2, axis=0) - hsm

    mag = jnp.sqrt(gx * gx + gy * gy)
    ri = lax.broadcasted_iota(jnp.int32, (_FR, _W), 0)
    rvalid = (ri >= 1) & (ri <= _H)
    mag = jnp.where(rvalid, mag * (1.0 / 64.0), 0.0)

    p = _atan2(gx, gy) * (_NBINS / math.pi)
    fl = jnp.floor(p)
    frac = p - fl
    bq = jnp.where(fl >= 10.0, fl - 10.0, jnp.where(fl < 0.0, fl + 10.0, fl))
    tq = jnp.where(frac == 0.0, bq, jnp.where(bq == 9.0, 0.0, bq + 1.0))
    b_v = mag * (1.0 - frac)
    t_v = mag * ((1.0 + frac) - (tq - bq))

    bqh = bq.astype(jnp.bfloat16)
    tqh = tq.astype(jnp.bfloat16)
    bvh = b_v.astype(jnp.bfloat16)
    tvh = t_v.astype(jnp.bfloat16)
    zero = jnp.zeros_like(bvh)

    bc = lax.broadcasted_iota(jnp.int32, (_W, 512), 0)
    bj = lax.broadcasted_iota(jnp.int32, (_W, 512), 1)
    bmat = ((bc - bj >= -1) & (bc - bj <= 6)).astype(jnp.bfloat16)
    for k in range(_NBINS):
        kk = jnp.bfloat16(k)
        h = jnp.where(bqh == kk, bvh, zero) + jnp.where(tqh == kk, tvh, zero)
        h = h + jnp.roll(h, -1, axis=0)
        h = h + jnp.roll(h, -2, axis=0)
        h = h + jnp.roll(h, -4, axis=0)
        pooled = jnp.dot(h, bmat, preferred_element_type=jnp.float32)
        o_ref[0, k] = pooled[0:_OUT, 0:_OUT]


def kernel(x):
    n = x.shape[0]
    return pl.pallas_call(
        _hog_body,
        grid=(n,),
        in_specs=[pl.BlockSpec((1, 1, _H, _W), lambda i: (i, 0, 0, 0))],
        out_specs=pl.BlockSpec((1, _NBINS, _OUT, _OUT), lambda i: (i, 0, 0, 0)),
        out_shape=jax.ShapeDtypeStruct((n, _NBINS, _OUT, _OUT), jnp.float32),
    )(x)
